# Initial kernel scaffold; baseline (speedup 1.0000x reference)
import jax
import jax.numpy as jnp
from jax.experimental import pallas as pl


def kernel(x, edge_index, c_param, W_enc, b_enc, W0, b0, W1, b1, W2, b2, W_head, b_head):
    raise NotImplementedError("write your pallas kernel here")


# speedup vs baseline: 3.3945x; 3.3945x over previous
import functools

import jax
import jax.numpy as jnp
from jax import lax
from jax.experimental import pallas as pl
from jax.experimental.pallas import tpu as pltpu
from jax.experimental.pallas import tpu_sc as plsc

EPS = 1e-7

N = 10000
E = 160000
HID = 256
HALF = 128
D_OUT = 128

_NC = 2
_NS = 16
_LANES = 16

_ROW_BLK = 1024
_CHUNK = 80
_E_PER_TILE = E // _NS
_N_CHUNKS = _E_PER_TILE // _CHUNK
_OUT_BLK = 104
_OUT_PER_TILE = 6
_TAIL_ROWS = N - _NS * _OUT_PER_TILE * _OUT_BLK


def _row_norm(v):
    return jnp.sqrt(jnp.sum(v * v, axis=-1, keepdims=True))


def _exp_map(v, sc):
    n = jnp.maximum(_row_norm(v), EPS)
    return jnp.tanh(sc * n) * v / (sc * n)


def _arctanh(x):
    return 0.5 * jnp.log((1.0 + x) / (1.0 - x))


def _log_map(y, sc):
    n = jnp.maximum(_row_norm(y), EPS)
    scn = jnp.clip(sc * n, EPS, 1.0 - 1e-5)
    return _arctanh(scn) * y / (sc * n)


def _split_store(m_ref, m):
    m_ref[0] = m[:, :HALF]
    m_ref[1] = m[:, HALF:]


def _enc_body(sc_ref, x_ref, we_ref, be_ref, w0_ref, b0_ref, m_ref):
    sc = sc_ref[0, 0]
    y = jnp.dot(x_ref[...], we_ref[...],
                preferred_element_type=jnp.float32) + be_ref[...]
    ht = _log_map(_exp_map(y, sc), sc)
    _split_store(m_ref, jnp.dot(ht, w0_ref[...],
                                preferred_element_type=jnp.float32) + b0_ref[...])


def _deg_col(deg_ref):
    d = deg_ref[0] + deg_ref[1]
    return jnp.maximum(d[:, 0:1], 1.0)


def _mid_body(sc_ref, agg_ref, deg_ref, w_ref, b_ref, m_ref):
    sc = sc_ref[0, 0]
    deg = _deg_col(deg_ref)
    z = agg_ref[...] / deg
    h = _exp_map(_log_map(_exp_map(z, sc), sc), sc)
    ht = _log_map(h, sc)
    _split_store(m_ref, jnp.dot(ht, w_ref[...],
                                preferred_element_type=jnp.float32) + b_ref[...])


def _head_body(sc_ref, agg_ref, deg_ref, w_ref, b_ref, o_ref):
    sc = sc_ref[0, 0]
    deg = _deg_col(deg_ref)
    z = agg_ref[...] / deg
    h = _exp_map(_log_map(_exp_map(z, sc), sc), sc)
    ht = _log_map(h, sc)
    o_ref[...] = jnp.dot(ht, w_ref[...],
                         preferred_element_type=jnp.float32) + b_ref[...]


_GRID = -(-N // _ROW_BLK)

_SC_SPEC = pl.BlockSpec((1, 1), lambda i: (0, 0))
_ROW_SPEC = pl.BlockSpec((_ROW_BLK, HID), lambda i: (i, 0))
_DEG_SPEC = pl.BlockSpec((2, _ROW_BLK, HALF), lambda i: (0, i, 0))
_W_SPEC = pl.BlockSpec((HID, HID), lambda i: (0, 0))
_B_SPEC = pl.BlockSpec((1, HID), lambda i: (0, 0))
_M_SPEC = pl.BlockSpec((2, _ROW_BLK, HALF), lambda i: (0, i, 0))
_M_SHAPE = jax.ShapeDtypeStruct((2, N, HALF), jnp.float32)


def _enc_call(sc, x, W_enc, b_enc, W0, b0):
    return pl.pallas_call(
        _enc_body,
        grid=(_GRID,),
        in_specs=[_SC_SPEC, _ROW_SPEC, _W_SPEC, _B_SPEC, _W_SPEC, _B_SPEC],
        out_specs=_M_SPEC,
        out_shape=_M_SHAPE,
    )(sc, x, W_enc, b_enc, W0, b0)


def _mid_call(sc, agg, deg, W, b):
    return pl.pallas_call(
        _mid_body,
        grid=(_GRID,),
        in_specs=[_SC_SPEC, _ROW_SPEC, _DEG_SPEC, _W_SPEC, _B_SPEC],
        out_specs=_M_SPEC,
        out_shape=_M_SHAPE,
    )(sc, agg, deg, W, b)


def _head_call(sc, agg, deg, W_head, b_head):
    return pl.pallas_call(
        _head_body,
        grid=(_GRID,),
        in_specs=[_SC_SPEC, _ROW_SPEC, _DEG_SPEC,
                  pl.BlockSpec((HID, D_OUT), lambda i: (0, 0)),
                  pl.BlockSpec((1, D_OUT), lambda i: (0, 0))],
        out_specs=pl.BlockSpec((_ROW_BLK, D_OUT), lambda i: (i, 0)),
        out_shape=jax.ShapeDtypeStruct((N, D_OUT), jnp.float32),
    )(sc, agg, deg, W_head, b_head)


def _zero_fill(zbuf):
    zero16 = jnp.zeros((_LANES,), jnp.float32)
    for i in range(8):
        for j in range(HALF // _LANES):
            zbuf[i, pl.ds(j * _LANES, _LANES)] = zero16


def _for_tile_chunks(s, fn):
    for t in range(_OUT_PER_TILE):
        fn((s * _OUT_PER_TILE + t) * _OUT_BLK, _OUT_BLK)

    @pl.when(s == 0)
    def _():
        fn(_NS * _OUT_PER_TILE * _OUT_BLK, _TAIL_ROWS)


def _agg_body(m_hbm, src_hbm, dst_hbm, out_hbm,
              srcv, dstv, gbuf, zbuf, aggs, sem):
    c = lax.axis_index("c")
    s = lax.axis_index("s")

    _zero_fill(zbuf)

    def zero_rows(r0, nrows):
        for q in range(nrows // 8):
            pltpu.sync_copy(zbuf, aggs.at[pl.ds(r0 + q * 8, 8)])

    _for_tile_chunks(s, zero_rows)
    plsc.subcore_barrier()

    ebase = s * _E_PER_TILE
    coff = c * N

    def step(k, carry):
        eoff = ebase + k * _CHUNK
        pltpu.sync_copy(src_hbm.at[pl.ds(eoff, _CHUNK)], srcv)
        pltpu.sync_copy(dst_hbm.at[pl.ds(eoff, _CHUNK)], dstv)
        for i in range(_CHUNK // _LANES):
            sl = pl.ds(i * _LANES, _LANES)
            srcv[sl] = srcv[sl] + coff
        pltpu.async_copy(m_hbm.at[srcv], gbuf, sem).wait()
        pltpu.sync_copy(gbuf, aggs.at[dstv], add=True)
        return carry

    lax.fori_loop(0, _N_CHUNKS, step, 0)
    plsc.subcore_barrier()

    def write_rows(r0, nrows):
        pltpu.sync_copy(aggs.at[pl.ds(r0, nrows)],
                        out_hbm.at[pl.ds(r0, nrows), pl.ds(c * HALF, HALF)])

    _for_tile_chunks(s, write_rows)


_DCHUNK = 40
_E_PER_WORKER = E // (_NC * _NS)
_D_CHUNKS = _E_PER_WORKER // _DCHUNK


def _deg_body(dst_hbm, deg_hbm, dstv, onesb, zbuf, degs, sem):
    c = lax.axis_index("c")
    s = lax.axis_index("s")

    _zero_fill(zbuf)
    one16 = jnp.ones((_LANES,), jnp.float32)

    def ofill(i, carry):
        for j in range(HALF // _LANES):
            onesb[i, pl.ds(j * _LANES, _LANES)] = one16
        return carry

    lax.fori_loop(0, _DCHUNK, ofill, 0)

    def zero_rows(r0, nrows):
        for q in range(nrows // 8):
            pltpu.sync_copy(zbuf, degs.at[pl.ds(r0 + q * 8, 8)])

    _for_tile_chunks(s, zero_rows)
    plsc.subcore_barrier()

    ebase = (c * _NS + s) * _E_PER_WORKER

    def step(k, carry):
        pltpu.sync_copy(dst_hbm.at[pl.ds(ebase + k * _DCHUNK, _DCHUNK)], dstv)
        pltpu.sync_copy(onesb, degs.at[dstv], add=True)
        return carry

    lax.fori_loop(0, _D_CHUNKS, step, 0)
    plsc.subcore_barrier()

    def write_rows(r0, nrows):
        pltpu.sync_copy(degs.at[pl.ds(r0, nrows)],
                        deg_hbm.at[pl.ds(c * N + r0, nrows)])

    _for_tile_chunks(s, write_rows)


@functools.lru_cache(maxsize=None)
def _make_agg():
    mesh = plsc.VectorSubcoreMesh(core_axis_name="c", subcore_axis_name="s",
                                  num_cores=_NC, num_subcores=_NS)
    scratch = [
        pltpu.VMEM((_CHUNK,), jnp.int32),
        pltpu.VMEM((_CHUNK,), jnp.int32),
        pltpu.VMEM((_CHUNK, HALF), jnp.float32),
        pltpu.VMEM((8, HALF), jnp.float32),
        pltpu.VMEM_SHARED((N, HALF), jnp.float32),
        pltpu.SemaphoreType.DMA,
    ]
    return pl.kernel(_agg_body,
                     out_type=jax.ShapeDtypeStruct((N, HID), jnp.float32),
                     mesh=mesh, scratch_types=tuple(scratch))


@functools.lru_cache(maxsize=None)
def _make_deg():
    mesh = plsc.VectorSubcoreMesh(core_axis_name="c", subcore_axis_name="s",
                                  num_cores=_NC, num_subcores=_NS)
    scratch = [
        pltpu.VMEM((_DCHUNK,), jnp.int32),
        pltpu.VMEM((_DCHUNK, HALF), jnp.float32),
        pltpu.VMEM((8, HALF), jnp.float32),
        pltpu.VMEM_SHARED((N, HALF), jnp.float32),
        pltpu.SemaphoreType.DMA,
    ]
    return pl.kernel(_deg_body,
                     out_type=jax.ShapeDtypeStruct((2 * N, HALF), jnp.float32),
                     mesh=mesh, scratch_types=tuple(scratch))


def kernel(x, edge_index, c_param, W_enc, b_enc, W0, b0, W1, b1, W2, b2,
           W_head, b_head):
    c = jnp.abs(c_param) + 1e-5
    sc = jnp.sqrt(c).reshape(1, 1).astype(jnp.float32)

    ei = edge_index.astype(jnp.int32)
    src = ei[0]
    dst = ei[1]

    deg = _make_deg()(dst).reshape(2, N, HALF)
    m = _enc_call(sc, x, W_enc, b_enc.reshape(1, HID), W0, b0.reshape(1, HID))
    agg = _make_agg()(m.reshape(2 * N, HALF), src, dst)

    for (W, b) in ((W1, b1), (W2, b2)):
        m = _mid_call(sc, agg, deg, W, b.reshape(1, HID))
        agg = _make_agg()(m.reshape(2 * N, HALF), src, dst)

    return _head_call(sc, agg, deg, W_head, b_head.reshape(1, D_OUT))


# speedup vs baseline: 5.0978x; 1.5018x over previous
import functools

import jax
import jax.numpy as jnp
from jax import lax
from jax.experimental import pallas as pl
from jax.experimental.pallas import tpu as pltpu
from jax.experimental.pallas import tpu_sc as plsc

EPS = 1e-7

N = 10000
E = 160000
HID = 256
HALF = 128
D_OUT = 128

_NC = 2
_NS = 16
_LANES = 16

_ROW_BLK = 1024
_CHUNK = 80
_E_PER_TILE = E // _NS
_N_CHUNKS = _E_PER_TILE // _CHUNK
_OUT_BLK = 104
_OUT_PER_TILE = 6
_TAIL_ROWS = N - _NS * _OUT_PER_TILE * _OUT_BLK


def _row_norm(v):
    return jnp.sqrt(jnp.sum(v * v, axis=-1, keepdims=True))


def _exp_map(v, sc):
    n = jnp.maximum(_row_norm(v), EPS)
    return jnp.tanh(sc * n) * v / (sc * n)


def _arctanh(x):
    return 0.5 * jnp.log((1.0 + x) / (1.0 - x))


def _log_map(y, sc):
    n = jnp.maximum(_row_norm(y), EPS)
    scn = jnp.clip(sc * n, EPS, 1.0 - 1e-5)
    return _arctanh(scn) * y / (sc * n)


def _split_store(m_ref, m):
    m_ref[0] = m[:, :HALF]
    m_ref[1] = m[:, HALF:]


def _enc_body(sc_ref, x_ref, we_ref, be_ref, w0_ref, b0_ref, m_ref):
    sc = sc_ref[0, 0]
    y = jnp.dot(x_ref[...], we_ref[...],
                preferred_element_type=jnp.float32) + be_ref[...]
    ht = _log_map(_exp_map(y, sc), sc)
    _split_store(m_ref, jnp.dot(ht, w0_ref[...],
                                preferred_element_type=jnp.float32) + b0_ref[...])


def _deg_col(deg_ref):
    d = deg_ref[0] + deg_ref[1]
    return jnp.maximum(d[:, 0:1], 1.0)


def _mid_body(sc_ref, agg_ref, deg_ref, w_ref, b_ref, m_ref):
    sc = sc_ref[0, 0]
    deg = _deg_col(deg_ref)
    z = agg_ref[...] / deg
    h = _exp_map(_log_map(_exp_map(z, sc), sc), sc)
    ht = _log_map(h, sc)
    _split_store(m_ref, jnp.dot(ht, w_ref[...],
                                preferred_element_type=jnp.float32) + b_ref[...])


def _head_body(sc_ref, agg_ref, deg_ref, w_ref, b_ref, o_ref):
    sc = sc_ref[0, 0]
    deg = _deg_col(deg_ref)
    z = agg_ref[...] / deg
    h = _exp_map(_log_map(_exp_map(z, sc), sc), sc)
    ht = _log_map(h, sc)
    o_ref[...] = jnp.dot(ht, w_ref[...],
                         preferred_element_type=jnp.float32) + b_ref[...]


_GRID = -(-N // _ROW_BLK)

_SC_SPEC = pl.BlockSpec((1, 1), lambda i: (0, 0))
_ROW_SPEC = pl.BlockSpec((_ROW_BLK, HID), lambda i: (i, 0))
_DEG_SPEC = pl.BlockSpec((2, _ROW_BLK, HALF), lambda i: (0, i, 0))
_W_SPEC = pl.BlockSpec((HID, HID), lambda i: (0, 0))
_B_SPEC = pl.BlockSpec((1, HID), lambda i: (0, 0))
_M_SPEC = pl.BlockSpec((2, _ROW_BLK, HALF), lambda i: (0, i, 0))
_M_SHAPE = jax.ShapeDtypeStruct((2, N, HALF), jnp.float32)


def _enc_call(sc, x, W_enc, b_enc, W0, b0):
    return pl.pallas_call(
        _enc_body,
        grid=(_GRID,),
        in_specs=[_SC_SPEC, _ROW_SPEC, _W_SPEC, _B_SPEC, _W_SPEC, _B_SPEC],
        out_specs=_M_SPEC,
        out_shape=_M_SHAPE,
    )(sc, x, W_enc, b_enc, W0, b0)


def _mid_call(sc, agg, deg, W, b):
    return pl.pallas_call(
        _mid_body,
        grid=(_GRID,),
        in_specs=[_SC_SPEC, _ROW_SPEC, _DEG_SPEC, _W_SPEC, _B_SPEC],
        out_specs=_M_SPEC,
        out_shape=_M_SHAPE,
    )(sc, agg, deg, W, b)


def _head_call(sc, agg, deg, W_head, b_head):
    return pl.pallas_call(
        _head_body,
        grid=(_GRID,),
        in_specs=[_SC_SPEC, _ROW_SPEC, _DEG_SPEC,
                  pl.BlockSpec((HID, D_OUT), lambda i: (0, 0)),
                  pl.BlockSpec((1, D_OUT), lambda i: (0, 0))],
        out_specs=pl.BlockSpec((_ROW_BLK, D_OUT), lambda i: (i, 0)),
        out_shape=jax.ShapeDtypeStruct((N, D_OUT), jnp.float32),
    )(sc, agg, deg, W_head, b_head)


def _zero_fill(zbuf):
    zero16 = jnp.zeros((_LANES,), jnp.float32)
    for i in range(8):
        for j in range(HALF // _LANES):
            zbuf[i, pl.ds(j * _LANES, _LANES)] = zero16


def _for_tile_chunks(s, fn):
    for t in range(_OUT_PER_TILE):
        fn((s * _OUT_PER_TILE + t) * _OUT_BLK, _OUT_BLK)

    @pl.when(s == 0)
    def _():
        fn(_NS * _OUT_PER_TILE * _OUT_BLK, _TAIL_ROWS)


def _agg_body(m_hbm, src_hbm, dst_hbm, out_hbm,
              srcva, dstva, gbufa, srcvb, dstvb, gbufb, zbuf, aggs,
              sema, semb):
    c = lax.axis_index("c")
    s = lax.axis_index("s")

    _zero_fill(zbuf)

    def zero_rows(r0, nrows):
        for q in range(nrows // 8):
            pltpu.sync_copy(zbuf, aggs.at[pl.ds(r0 + q * 8, 8)])

    _for_tile_chunks(s, zero_rows)
    plsc.subcore_barrier()

    ebase = s * _E_PER_TILE
    coff = c * N

    def start_gather(k, srcv, dstv, gbuf, sem):
        eoff = ebase + k * _CHUNK
        pltpu.sync_copy(src_hbm.at[pl.ds(eoff, _CHUNK)], srcv)
        pltpu.sync_copy(dst_hbm.at[pl.ds(eoff, _CHUNK)], dstv)
        for i in range(_CHUNK // _LANES):
            sl = pl.ds(i * _LANES, _LANES)
            srcv[sl] = srcv[sl] + coff
        return pltpu.async_copy(m_hbm.at[srcv], gbuf, sem)

    start_gather(0, srcva, dstva, gbufa, sema)

    def steppair(k2, carry):
        start_gather(2 * k2 + 1, srcvb, dstvb, gbufb, semb)
        pltpu.make_async_copy(m_hbm.at[srcva], gbufa, sema).wait()
        pltpu.sync_copy(gbufa, aggs.at[dstva], add=True)
        start_gather(2 * k2 + 2, srcva, dstva, gbufa, sema)
        pltpu.make_async_copy(m_hbm.at[srcvb], gbufb, semb).wait()
        pltpu.sync_copy(gbufb, aggs.at[dstvb], add=True)
        return carry

    lax.fori_loop(0, (_N_CHUNKS - 1) // 2, steppair, 0)
    pltpu.make_async_copy(m_hbm.at[srcva], gbufa, sema).wait()
    pltpu.sync_copy(gbufa, aggs.at[dstva], add=True)
    plsc.subcore_barrier()

    def write_rows(r0, nrows):
        pltpu.sync_copy(aggs.at[pl.ds(r0, nrows)],
                        out_hbm.at[pl.ds(r0, nrows), pl.ds(c * HALF, HALF)])

    _for_tile_chunks(s, write_rows)


_DCHUNK = 40
_E_PER_WORKER = E // (_NC * _NS)
_D_CHUNKS = _E_PER_WORKER // _DCHUNK


def _deg_body(dst_hbm, deg_hbm, dstv, onesb, zbuf, degs, sem):
    c = lax.axis_index("c")
    s = lax.axis_index("s")

    _zero_fill(zbuf)
    one16 = jnp.ones((_LANES,), jnp.float32)

    def ofill(i, carry):
        for j in range(HALF // _LANES):
            onesb[i, pl.ds(j * _LANES, _LANES)] = one16
        return carry

    lax.fori_loop(0, _DCHUNK, ofill, 0)

    def zero_rows(r0, nrows):
        for q in range(nrows // 8):
            pltpu.sync_copy(zbuf, degs.at[pl.ds(r0 + q * 8, 8)])

    _for_tile_chunks(s, zero_rows)
    plsc.subcore_barrier()

    ebase = (c * _NS + s) * _E_PER_WORKER

    def step(k, carry):
        pltpu.sync_copy(dst_hbm.at[pl.ds(ebase + k * _DCHUNK, _DCHUNK)], dstv)
        pltpu.sync_copy(onesb, degs.at[dstv], add=True)
        return carry

    lax.fori_loop(0, _D_CHUNKS, step, 0)
    plsc.subcore_barrier()

    def write_rows(r0, nrows):
        pltpu.sync_copy(degs.at[pl.ds(r0, nrows)],
                        deg_hbm.at[pl.ds(c * N + r0, nrows)])

    _for_tile_chunks(s, write_rows)


@functools.lru_cache(maxsize=None)
def _make_agg():
    mesh = plsc.VectorSubcoreMesh(core_axis_name="c", subcore_axis_name="s",
                                  num_cores=_NC, num_subcores=_NS)
    scratch = [
        pltpu.VMEM((_CHUNK,), jnp.int32),
        pltpu.VMEM((_CHUNK,), jnp.int32),
        pltpu.VMEM((_CHUNK, HALF), jnp.float32),
        pltpu.VMEM((_CHUNK,), jnp.int32),
        pltpu.VMEM((_CHUNK,), jnp.int32),
        pltpu.VMEM((_CHUNK, HALF), jnp.float32),
        pltpu.VMEM((8, HALF), jnp.float32),
        pltpu.VMEM_SHARED((N, HALF), jnp.float32),
        pltpu.SemaphoreType.DMA,
        pltpu.SemaphoreType.DMA,
    ]
    return pl.kernel(_agg_body,
                     out_type=jax.ShapeDtypeStruct((N, HID), jnp.float32),
                     mesh=mesh, scratch_types=tuple(scratch))


@functools.lru_cache(maxsize=None)
def _make_deg():
    mesh = plsc.VectorSubcoreMesh(core_axis_name="c", subcore_axis_name="s",
                                  num_cores=_NC, num_subcores=_NS)
    scratch = [
        pltpu.VMEM((_DCHUNK,), jnp.int32),
        pltpu.VMEM((_DCHUNK, HALF), jnp.float32),
        pltpu.VMEM((8, HALF), jnp.float32),
        pltpu.VMEM_SHARED((N, HALF), jnp.float32),
        pltpu.SemaphoreType.DMA,
    ]
    return pl.kernel(_deg_body,
                     out_type=jax.ShapeDtypeStruct((2 * N, HALF), jnp.float32),
                     mesh=mesh, scratch_types=tuple(scratch))


def kernel(x, edge_index, c_param, W_enc, b_enc, W0, b0, W1, b1, W2, b2,
           W_head, b_head):
    c = jnp.abs(c_param) + 1e-5
    sc = jnp.sqrt(c).reshape(1, 1).astype(jnp.float32)

    ei = edge_index.astype(jnp.int32)
    src = ei[0]
    dst = ei[1]

    deg = _make_deg()(dst).reshape(2, N, HALF)
    m = _enc_call(sc, x, W_enc, b_enc.reshape(1, HID), W0, b0.reshape(1, HID))
    agg = _make_agg()(m.reshape(2 * N, HALF), src, dst)

    for (W, b) in ((W1, b1), (W2, b2)):
        m = _mid_call(sc, agg, deg, W, b.reshape(1, HID))
        agg = _make_agg()(m.reshape(2 * N, HALF), src, dst)

    return _head_call(sc, agg, deg, W_head, b_head.reshape(1, D_OUT))


# speedup vs baseline: 7.0353x; 1.3801x over previous
import functools

import jax
import jax.numpy as jnp
from jax import lax
from jax.experimental import pallas as pl
from jax.experimental.pallas import tpu as pltpu
from jax.experimental.pallas import tpu_sc as plsc

EPS = 1e-7

N = 10000
E = 160000
HID = 256
HALF = 128
D_OUT = 128

_NC = 2
_NS = 16
_LANES = 16

_ROW_BLK = 1024
_CHUNK = 80
_NB = 25
_E_PER_TILE = E // _NS
_N_CHUNKS = _E_PER_TILE // _CHUNK
_OUT_BLK = 104
_OUT_PER_TILE = 6
_TAIL_ROWS = N - _NS * _OUT_PER_TILE * _OUT_BLK


def _row_norm(v):
    return jnp.sqrt(jnp.sum(v * v, axis=-1, keepdims=True))


def _exp_map(v, sc):
    n = jnp.maximum(_row_norm(v), EPS)
    return jnp.tanh(sc * n) * v / (sc * n)


def _arctanh(x):
    return 0.5 * jnp.log((1.0 + x) / (1.0 - x))


def _log_map(y, sc):
    n = jnp.maximum(_row_norm(y), EPS)
    scn = jnp.clip(sc * n, EPS, 1.0 - 1e-5)
    return _arctanh(scn) * y / (sc * n)


def _split_store(m_ref, m):
    m_ref[0] = m[:, :HALF]
    m_ref[1] = m[:, HALF:]


def _enc_body(sc_ref, x_ref, we_ref, be_ref, w0_ref, b0_ref, m_ref):
    sc = sc_ref[0, 0]
    y = jnp.dot(x_ref[...], we_ref[...],
                preferred_element_type=jnp.float32) + be_ref[...]
    ht = _log_map(_exp_map(y, sc), sc)
    _split_store(m_ref, jnp.dot(ht, w0_ref[...],
                                preferred_element_type=jnp.float32) + b0_ref[...])


def _deg_col(deg_ref):
    d = deg_ref[0] + deg_ref[1]
    return jnp.maximum(d[:, 0:1], 1.0)


def _mid_body(sc_ref, agg_ref, deg_ref, w_ref, b_ref, m_ref):
    sc = sc_ref[0, 0]
    deg = _deg_col(deg_ref)
    z = agg_ref[...] / deg
    h = _exp_map(_log_map(_exp_map(z, sc), sc), sc)
    ht = _log_map(h, sc)
    _split_store(m_ref, jnp.dot(ht, w_ref[...],
                                preferred_element_type=jnp.float32) + b_ref[...])


def _head_body(sc_ref, agg_ref, deg_ref, w_ref, b_ref, o_ref):
    sc = sc_ref[0, 0]
    deg = _deg_col(deg_ref)
    z = agg_ref[...] / deg
    h = _exp_map(_log_map(_exp_map(z, sc), sc), sc)
    ht = _log_map(h, sc)
    o_ref[...] = jnp.dot(ht, w_ref[...],
                         preferred_element_type=jnp.float32) + b_ref[...]


_GRID = -(-N // _ROW_BLK)

_SC_SPEC = pl.BlockSpec((1, 1), lambda i: (0, 0))
_ROW_SPEC = pl.BlockSpec((_ROW_BLK, HID), lambda i: (i, 0))
_DEG_SPEC = pl.BlockSpec((2, _ROW_BLK, HALF), lambda i: (0, i, 0))
_W_SPEC = pl.BlockSpec((HID, HID), lambda i: (0, 0))
_B_SPEC = pl.BlockSpec((1, HID), lambda i: (0, 0))
_M_SPEC = pl.BlockSpec((2, _ROW_BLK, HALF), lambda i: (0, i, 0))
_M_SHAPE = jax.ShapeDtypeStruct((2, N, HALF), jnp.float32)


def _enc_call(sc, x, W_enc, b_enc, W0, b0):
    return pl.pallas_call(
        _enc_body,
        grid=(_GRID,),
        in_specs=[_SC_SPEC, _ROW_SPEC, _W_SPEC, _B_SPEC, _W_SPEC, _B_SPEC],
        out_specs=_M_SPEC,
        out_shape=_M_SHAPE,
    )(sc, x, W_enc, b_enc, W0, b0)


def _mid_call(sc, agg, deg, W, b):
    return pl.pallas_call(
        _mid_body,
        grid=(_GRID,),
        in_specs=[_SC_SPEC, _ROW_SPEC, _DEG_SPEC, _W_SPEC, _B_SPEC],
        out_specs=_M_SPEC,
        out_shape=_M_SHAPE,
    )(sc, agg, deg, W, b)


def _head_call(sc, agg, deg, W_head, b_head):
    return pl.pallas_call(
        _head_body,
        grid=(_GRID,),
        in_specs=[_SC_SPEC, _ROW_SPEC, _DEG_SPEC,
                  pl.BlockSpec((HID, D_OUT), lambda i: (0, 0)),
                  pl.BlockSpec((1, D_OUT), lambda i: (0, 0))],
        out_specs=pl.BlockSpec((_ROW_BLK, D_OUT), lambda i: (i, 0)),
        out_shape=jax.ShapeDtypeStruct((N, D_OUT), jnp.float32),
    )(sc, agg, deg, W_head, b_head)


def _zero_fill(zbuf):
    zero16 = jnp.zeros((_LANES,), jnp.float32)
    for i in range(8):
        for j in range(HALF // _LANES):
            zbuf[i, pl.ds(j * _LANES, _LANES)] = zero16


def _for_tile_chunks(s, fn):
    for t in range(_OUT_PER_TILE):
        fn((s * _OUT_PER_TILE + t) * _OUT_BLK, _OUT_BLK)

    @pl.when(s == 0)
    def _():
        fn(_NS * _OUT_PER_TILE * _OUT_BLK, _TAIL_ROWS)


def _agg_body(m_hbm, src_hbm, dst_hbm, out_hbm,
              sidxb, didxb, dstv, gbufa, gbufb, zbuf, aggs,
              sema, semb):
    c = lax.axis_index("c")
    s = lax.axis_index("s")

    _zero_fill(zbuf)

    def zero_rows(r0, nrows):
        for q in range(nrows // 8):
            pltpu.sync_copy(zbuf, aggs.at[pl.ds(r0 + q * 8, 8)])

    _for_tile_chunks(s, zero_rows)
    plsc.subcore_barrier()

    ebase = s * _E_PER_TILE
    coff = c * N
    blen = _NB * _CHUNK

    def batch(b, carry):
        boff = ebase + b * blen
        pltpu.sync_copy(src_hbm.at[pl.ds(boff, blen)], sidxb)
        pltpu.sync_copy(dst_hbm.at[pl.ds(boff, blen)], didxb)
        for i in range(blen // _LANES):
            sl = pl.ds(i * _LANES, _LANES)
            sidxb[sl] = sidxb[sl] + coff

        def sg(j, gbuf, sem):
            idx = sidxb.at[pl.ds(j * _CHUNK, _CHUNK)]
            return pltpu.async_copy(m_hbm.at[idx], gbuf, sem)

        def scat(j, gbuf, sem):
            idx = sidxb.at[pl.ds(j * _CHUNK, _CHUNK)]
            pltpu.make_async_copy(m_hbm.at[idx], gbuf, sem).wait()
            for q in range(_CHUNK // _LANES):
                dstv[pl.ds(q * _LANES, _LANES)] = (
                    didxb[pl.ds(j * _CHUNK + q * _LANES, _LANES)])
            pltpu.sync_copy(gbuf, aggs.at[dstv], add=True)

        sg(0, gbufa, sema)
        for p in range(_NB // 2):
            sg(2 * p + 1, gbufb, semb)
            scat(2 * p, gbufa, sema)
            sg(2 * p + 2, gbufa, sema)
            scat(2 * p + 1, gbufb, semb)
        scat(_NB - 1, gbufa, sema)
        return carry

    lax.fori_loop(0, _N_CHUNKS // _NB, batch, 0)
    plsc.subcore_barrier()

    def write_rows(r0, nrows):
        pltpu.sync_copy(aggs.at[pl.ds(r0, nrows)],
                        out_hbm.at[pl.ds(r0, nrows), pl.ds(c * HALF, HALF)])

    _for_tile_chunks(s, write_rows)


_DCHUNK = 200
_E_PER_WORKER = E // (_NC * _NS)
_D_CHUNKS = _E_PER_WORKER // _DCHUNK


def _deg_body(dst_hbm, deg_hbm, dstv, onesb, zbuf, degs, sem):
    c = lax.axis_index("c")
    s = lax.axis_index("s")

    _zero_fill(zbuf)
    one16 = jnp.ones((_LANES,), jnp.float32)

    def ofill(i, carry):
        for j in range(HALF // _LANES):
            onesb[i, pl.ds(j * _LANES, _LANES)] = one16
        return carry

    lax.fori_loop(0, _DCHUNK, ofill, 0)

    def zero_rows(r0, nrows):
        for q in range(nrows // 8):
            pltpu.sync_copy(zbuf, degs.at[pl.ds(r0 + q * 8, 8)])

    _for_tile_chunks(s, zero_rows)
    plsc.subcore_barrier()

    ebase = (c * _NS + s) * _E_PER_WORKER

    def step(k, carry):
        pltpu.sync_copy(dst_hbm.at[pl.ds(ebase + k * _DCHUNK, _DCHUNK)], dstv)
        pltpu.sync_copy(onesb, degs.at[dstv], add=True)
        return carry

    lax.fori_loop(0, _D_CHUNKS, step, 0)
    plsc.subcore_barrier()

    def write_rows(r0, nrows):
        pltpu.sync_copy(degs.at[pl.ds(r0, nrows)],
                        deg_hbm.at[pl.ds(c * N + r0, nrows)])

    _for_tile_chunks(s, write_rows)


@functools.lru_cache(maxsize=None)
def _make_agg():
    mesh = plsc.VectorSubcoreMesh(core_axis_name="c", subcore_axis_name="s",
                                  num_cores=_NC, num_subcores=_NS)
    scratch = [
        pltpu.VMEM((_NB * _CHUNK,), jnp.int32),
        pltpu.VMEM((_NB * _CHUNK,), jnp.int32),
        pltpu.VMEM((_CHUNK,), jnp.int32),
        pltpu.VMEM((_CHUNK, HALF), jnp.float32),
        pltpu.VMEM((_CHUNK, HALF), jnp.float32),
        pltpu.VMEM((8, HALF), jnp.float32),
        pltpu.VMEM_SHARED((N, HALF), jnp.float32),
        pltpu.SemaphoreType.DMA,
        pltpu.SemaphoreType.DMA,
    ]
    return pl.kernel(_agg_body,
                     out_type=jax.ShapeDtypeStruct((N, HID), jnp.float32),
                     mesh=mesh, scratch_types=tuple(scratch))


@functools.lru_cache(maxsize=None)
def _make_deg():
    mesh = plsc.VectorSubcoreMesh(core_axis_name="c", subcore_axis_name="s",
                                  num_cores=_NC, num_subcores=_NS)
    scratch = [
        pltpu.VMEM((_DCHUNK,), jnp.int32),
        pltpu.VMEM((_DCHUNK, HALF), jnp.float32),
        pltpu.VMEM((8, HALF), jnp.float32),
        pltpu.VMEM_SHARED((N, HALF), jnp.float32),
        pltpu.SemaphoreType.DMA,
    ]
    return pl.kernel(_deg_body,
                     out_type=jax.ShapeDtypeStruct((2 * N, HALF), jnp.float32),
                     mesh=mesh, scratch_types=tuple(scratch))


def kernel(x, edge_index, c_param, W_enc, b_enc, W0, b0, W1, b1, W2, b2,
           W_head, b_head):
    c = jnp.abs(c_param) + 1e-5
    sc = jnp.sqrt(c).reshape(1, 1).astype(jnp.float32)

    ei = edge_index.astype(jnp.int32)
    src = ei[0]
    dst = ei[1]

    deg = _make_deg()(dst).reshape(2, N, HALF)
    m = _enc_call(sc, x, W_enc, b_enc.reshape(1, HID), W0, b0.reshape(1, HID))
    agg = _make_agg()(m.reshape(2 * N, HALF), src, dst)

    for (W, b) in ((W1, b1), (W2, b2)):
        m = _mid_call(sc, agg, deg, W, b.reshape(1, HID))
        agg = _make_agg()(m.reshape(2 * N, HALF), src, dst)

    return _head_call(sc, agg, deg, W_head, b_head.reshape(1, D_OUT))


# speedup vs baseline: 7.1734x; 1.0196x over previous
import functools

import jax
import jax.numpy as jnp
from jax import lax
from jax.experimental import pallas as pl
from jax.experimental.pallas import tpu as pltpu
from jax.experimental.pallas import tpu_sc as plsc

EPS = 1e-7

N = 10000
E = 160000
HID = 256
HALF = 128
D_OUT = 128

_NC = 2
_NS = 16
_LANES = 16

_ROW_BLK = 1024
_CHUNK = 80
_NB = 25
_E_PER_TILE = E // _NS
_N_CHUNKS = _E_PER_TILE // _CHUNK
_OUT_BLK = 208
_OUT_PER_TILE = 3
_TAIL_ROWS = N - _NS * _OUT_PER_TILE * _OUT_BLK
_ZROWS = 16
_DW = 128


def _row_norm(v):
    return jnp.sqrt(jnp.sum(v * v, axis=-1, keepdims=True))


def _exp_map(v, sc):
    n = jnp.maximum(_row_norm(v), EPS)
    return jnp.tanh(sc * n) * v / (sc * n)


def _arctanh(x):
    return 0.5 * jnp.log((1.0 + x) / (1.0 - x))


def _log_map(y, sc):
    n = jnp.maximum(_row_norm(y), EPS)
    scn = jnp.clip(sc * n, EPS, 1.0 - 1e-5)
    return _arctanh(scn) * y / (sc * n)


def _split_store(m_ref, m):
    m_ref[0] = m[:, :HALF]
    m_ref[1] = m[:, HALF:]


def _enc_body(sc_ref, x_ref, we_ref, be_ref, w0_ref, b0_ref, m_ref):
    sc = sc_ref[0, 0]
    y = jnp.dot(x_ref[...], we_ref[...],
                preferred_element_type=jnp.float32) + be_ref[...]
    ht = _log_map(_exp_map(y, sc), sc)
    _split_store(m_ref, jnp.dot(ht, w0_ref[...],
                                preferred_element_type=jnp.float32) + b0_ref[...])


def _deg_col(deg_ref):
    d = deg_ref[0] + deg_ref[1]
    return jnp.maximum(d[:, 0:1], 1.0)


def _mid_body(sc_ref, agg_ref, deg_ref, w_ref, b_ref, m_ref):
    sc = sc_ref[0, 0]
    deg = _deg_col(deg_ref)
    z = agg_ref[...].astype(jnp.float32) / deg
    h = _exp_map(_log_map(_exp_map(z, sc), sc), sc)
    ht = _log_map(h, sc)
    _split_store(m_ref, jnp.dot(ht, w_ref[...],
                                preferred_element_type=jnp.float32) + b_ref[...])


def _head_body(sc_ref, agg_ref, deg_ref, w_ref, b_ref, o_ref):
    sc = sc_ref[0, 0]
    deg = _deg_col(deg_ref)
    z = agg_ref[...].astype(jnp.float32) / deg
    h = _exp_map(_log_map(_exp_map(z, sc), sc), sc)
    ht = _log_map(h, sc)
    o_ref[...] = jnp.dot(ht, w_ref[...],
                         preferred_element_type=jnp.float32) + b_ref[...]


_GRID = -(-N // _ROW_BLK)

_SC_SPEC = pl.BlockSpec((1, 1), lambda i: (0, 0))
_ROW_SPEC = pl.BlockSpec((_ROW_BLK, HID), lambda i: (i, 0))
_DEG_SPEC = pl.BlockSpec((2, _ROW_BLK, _DW), lambda i: (0, i, 0))
_W_SPEC = pl.BlockSpec((HID, HID), lambda i: (0, 0))
_B_SPEC = pl.BlockSpec((1, HID), lambda i: (0, 0))
_M_SPEC = pl.BlockSpec((2, _ROW_BLK, HALF), lambda i: (0, i, 0))
_M_SHAPE = jax.ShapeDtypeStruct((2, N, HALF), jnp.float32)


def _enc_call(sc, x, W_enc, b_enc, W0, b0):
    return pl.pallas_call(
        _enc_body,
        grid=(_GRID,),
        in_specs=[_SC_SPEC, _ROW_SPEC, _W_SPEC, _B_SPEC, _W_SPEC, _B_SPEC],
        out_specs=_M_SPEC,
        out_shape=_M_SHAPE,
    )(sc, x, W_enc, b_enc, W0, b0)


def _mid_call(sc, agg, deg, W, b):
    return pl.pallas_call(
        _mid_body,
        grid=(_GRID,),
        in_specs=[_SC_SPEC, _ROW_SPEC, _DEG_SPEC, _W_SPEC, _B_SPEC],
        out_specs=_M_SPEC,
        out_shape=_M_SHAPE,
    )(sc, agg, deg, W, b)


def _head_call(sc, agg, deg, W_head, b_head):
    return pl.pallas_call(
        _head_body,
        grid=(_GRID,),
        in_specs=[_SC_SPEC, _ROW_SPEC, _DEG_SPEC,
                  pl.BlockSpec((HID, D_OUT), lambda i: (0, 0)),
                  pl.BlockSpec((1, D_OUT), lambda i: (0, 0))],
        out_specs=pl.BlockSpec((_ROW_BLK, D_OUT), lambda i: (i, 0)),
        out_shape=jax.ShapeDtypeStruct((N, D_OUT), jnp.float32),
    )(sc, agg, deg, W_head, b_head)


def _zero_fill(zbuf, nrows, ncols):
    zero16 = jnp.zeros((_LANES,), jnp.float32)

    def zrow(i, carry):
        for j in range(ncols // _LANES):
            zbuf[i, pl.ds(j * _LANES, _LANES)] = zero16
        return carry

    lax.fori_loop(0, nrows, zrow, 0)


def _zero_rows_of(zbuf, dstref):
    def zero_rows(r0, nrows):
        for q in range(nrows // _ZROWS):
            pltpu.sync_copy(zbuf, dstref.at[pl.ds(r0 + q * _ZROWS, _ZROWS)])
        rem = nrows % _ZROWS
        if rem:
            pltpu.sync_copy(zbuf.at[pl.ds(0, rem)],
                            dstref.at[pl.ds(r0 + nrows - rem, rem)])

    return zero_rows


def _for_tile_chunks(s, fn):
    for t in range(_OUT_PER_TILE):
        fn((s * _OUT_PER_TILE + t) * _OUT_BLK, _OUT_BLK)

    @pl.when(s == 0)
    def _():
        fn(_NS * _OUT_PER_TILE * _OUT_BLK, _TAIL_ROWS)


def _agg_body(m_hbm, src_hbm, dst_hbm, out_hbm,
              sidxb, didxb, dstv, gbufa, gbufb, zbuf, aggs,
              sema, semb):
    c = lax.axis_index("c")
    s = lax.axis_index("s")

    _zero_fill(zbuf, _ZROWS, HALF)
    _for_tile_chunks(s, _zero_rows_of(zbuf, aggs))
    plsc.subcore_barrier()

    ebase = s * _E_PER_TILE
    coff = c * N
    blen = _NB * _CHUNK

    def batch(b, carry):
        boff = ebase + b * blen
        pltpu.sync_copy(src_hbm.at[pl.ds(boff, blen)], sidxb)
        pltpu.sync_copy(dst_hbm.at[pl.ds(boff, blen)], didxb)
        for i in range(blen // _LANES):
            sl = pl.ds(i * _LANES, _LANES)
            sidxb[sl] = sidxb[sl] + coff

        def sg(j, gbuf, sem):
            idx = sidxb.at[pl.ds(j * _CHUNK, _CHUNK)]
            return pltpu.async_copy(m_hbm.at[idx], gbuf, sem)

        def scat(j, gbuf, sem):
            idx = sidxb.at[pl.ds(j * _CHUNK, _CHUNK)]
            pltpu.make_async_copy(m_hbm.at[idx], gbuf, sem).wait()
            for q in range(_CHUNK // _LANES):
                dstv[pl.ds(q * _LANES, _LANES)] = (
                    didxb[pl.ds(j * _CHUNK + q * _LANES, _LANES)])
            pltpu.sync_copy(gbuf, aggs.at[dstv], add=True)

        sg(0, gbufa, sema)
        for p in range(_NB // 2):
            sg(2 * p + 1, gbufb, semb)
            scat(2 * p, gbufa, sema)
            sg(2 * p + 2, gbufa, sema)
            scat(2 * p + 1, gbufb, semb)
        scat(_NB - 1, gbufa, sema)
        return carry

    lax.fori_loop(0, _N_CHUNKS // _NB, batch, 0)
    plsc.subcore_barrier()

    def write_rows(r0, nrows):
        pltpu.sync_copy(aggs.at[pl.ds(r0, nrows)],
                        out_hbm.at[pl.ds(r0, nrows), pl.ds(c * HALF, HALF)])

    _for_tile_chunks(s, write_rows)


_DCHUNK = 200
_E_PER_WORKER = E // (_NC * _NS)
_D_CHUNKS = _E_PER_WORKER // _DCHUNK


def _deg_body(dst_hbm, deg_hbm, dstv, onesb, zbuf, degs, sem):
    c = lax.axis_index("c")
    s = lax.axis_index("s")

    _zero_fill(zbuf, _ZROWS, _DW)
    one16 = jnp.ones((_LANES,), jnp.float32)

    def ofill(i, carry):
        for j in range(_DW // _LANES):
            onesb[i, pl.ds(j * _LANES, _LANES)] = one16
        return carry

    lax.fori_loop(0, _DCHUNK, ofill, 0)

    _for_tile_chunks(s, _zero_rows_of(zbuf, degs))
    plsc.subcore_barrier()

    ebase = (c * _NS + s) * _E_PER_WORKER

    def step(k, carry):
        pltpu.sync_copy(dst_hbm.at[pl.ds(ebase + k * _DCHUNK, _DCHUNK)], dstv)
        pltpu.sync_copy(onesb, degs.at[dstv], add=True)
        return carry

    lax.fori_loop(0, _D_CHUNKS, step, 0)
    plsc.subcore_barrier()

    def write_rows(r0, nrows):
        pltpu.sync_copy(degs.at[pl.ds(r0, nrows)],
                        deg_hbm.at[pl.ds(c * N + r0, nrows)])

    _for_tile_chunks(s, write_rows)


@functools.lru_cache(maxsize=None)
def _make_agg():
    mesh = plsc.VectorSubcoreMesh(core_axis_name="c", subcore_axis_name="s",
                                  num_cores=_NC, num_subcores=_NS)
    scratch = [
        pltpu.VMEM((_NB * _CHUNK,), jnp.int32),
        pltpu.VMEM((_NB * _CHUNK,), jnp.int32),
        pltpu.VMEM((_CHUNK,), jnp.int32),
        pltpu.VMEM((_CHUNK, HALF), jnp.float32),
        pltpu.VMEM((_CHUNK, HALF), jnp.float32),
        pltpu.VMEM((_ZROWS, HALF), jnp.float32),
        pltpu.VMEM_SHARED((N, HALF), jnp.float32),
        pltpu.SemaphoreType.DMA,
        pltpu.SemaphoreType.DMA,
    ]
    return pl.kernel(_agg_body,
                     out_type=jax.ShapeDtypeStruct((N, HID), jnp.float32),
                     mesh=mesh, scratch_types=tuple(scratch))


@functools.lru_cache(maxsize=None)
def _make_deg():
    mesh = plsc.VectorSubcoreMesh(core_axis_name="c", subcore_axis_name="s",
                                  num_cores=_NC, num_subcores=_NS)
    scratch = [
        pltpu.VMEM((_DCHUNK,), jnp.int32),
        pltpu.VMEM((_DCHUNK, _DW), jnp.float32),
        pltpu.VMEM((_ZROWS, _DW), jnp.float32),
        pltpu.VMEM_SHARED((N, _DW), jnp.float32),
        pltpu.SemaphoreType.DMA,
    ]
    return pl.kernel(_deg_body,
                     out_type=jax.ShapeDtypeStruct((2 * N, _DW), jnp.float32),
                     mesh=mesh, scratch_types=tuple(scratch))


def kernel(x, edge_index, c_param, W_enc, b_enc, W0, b0, W1, b1, W2, b2,
           W_head, b_head):
    c = jnp.abs(c_param) + 1e-5
    sc = jnp.sqrt(c).reshape(1, 1).astype(jnp.float32)

    ei = edge_index.astype(jnp.int32)
    src = ei[0]
    dst = ei[1]

    deg = _make_deg()(dst).reshape(2, N, _DW)
    m = _enc_call(sc, x, W_enc, b_enc.reshape(1, HID), W0, b0.reshape(1, HID))
    agg = _make_agg()(m.reshape(2 * N, HALF), src, dst)

    for (W, b) in ((W1, b1), (W2, b2)):
        m = _mid_call(sc, agg, deg, W, b.reshape(1, HID))
        agg = _make_agg()(m.reshape(2 * N, HALF), src, dst)

    return _head_call(sc, agg, deg, W_head, b_head.reshape(1, D_OUT))


# speedup vs baseline: 7.3551x; 1.0253x over previous
import functools

import jax
import jax.numpy as jnp
from jax import lax
from jax.experimental import pallas as pl
from jax.experimental.pallas import tpu as pltpu
from jax.experimental.pallas import tpu_sc as plsc

EPS = 1e-7

N = 10000
E = 160000
HID = 256
HALF = 128
D_OUT = 128

_NC = 2
_NS = 16
_LANES = 16

_ROW_BLK = 2048
_CHUNK = 80
_NB = 25
_E_PER_TILE = E // _NS
_N_CHUNKS = _E_PER_TILE // _CHUNK
_OUT_BLK = 208
_OUT_PER_TILE = 3
_TAIL_ROWS = N - _NS * _OUT_PER_TILE * _OUT_BLK
_ZROWS = 16
_DW = 128


def _row_norm(v):
    return jnp.sqrt(jnp.sum(v * v, axis=-1, keepdims=True))


def _exp_map(v, sc):
    n = jnp.maximum(_row_norm(v), EPS)
    return jnp.tanh(sc * n) * v / (sc * n)


def _arctanh(x):
    return 0.5 * jnp.log((1.0 + x) / (1.0 - x))


def _log_map(y, sc):
    n = jnp.maximum(_row_norm(y), EPS)
    scn = jnp.clip(sc * n, EPS, 1.0 - 1e-5)
    return _arctanh(scn) * y / (sc * n)


def _split_store(m_ref, m):
    m_ref[0] = m[:, :HALF]
    m_ref[1] = m[:, HALF:]


def _enc_body(sc_ref, x_ref, we_ref, be_ref, w0_ref, b0_ref, m_ref):
    sc = sc_ref[0, 0]
    y = jnp.dot(x_ref[...], we_ref[...],
                preferred_element_type=jnp.float32) + be_ref[...]
    ht = _log_map(_exp_map(y, sc), sc)
    _split_store(m_ref, jnp.dot(ht, w0_ref[...],
                                preferred_element_type=jnp.float32) + b0_ref[...])


def _deg_col(deg_ref):
    d = deg_ref[0] + deg_ref[1]
    return jnp.maximum(d[:, 0:1], 1.0)


def _mid_body(sc_ref, agg_ref, deg_ref, w_ref, b_ref, m_ref):
    sc = sc_ref[0, 0]
    deg = _deg_col(deg_ref)
    z = agg_ref[...].astype(jnp.float32) / deg
    h = _exp_map(_log_map(_exp_map(z, sc), sc), sc)
    ht = _log_map(h, sc)
    _split_store(m_ref, jnp.dot(ht, w_ref[...],
                                preferred_element_type=jnp.float32) + b_ref[...])


def _head_body(sc_ref, agg_ref, deg_ref, w_ref, b_ref, o_ref):
    sc = sc_ref[0, 0]
    deg = _deg_col(deg_ref)
    z = agg_ref[...].astype(jnp.float32) / deg
    h = _exp_map(_log_map(_exp_map(z, sc), sc), sc)
    ht = _log_map(h, sc)
    o_ref[...] = jnp.dot(ht, w_ref[...],
                         preferred_element_type=jnp.float32) + b_ref[...]


_GRID = -(-N // _ROW_BLK)

_SC_SPEC = pl.BlockSpec((1, 1), lambda i: (0, 0))
_ROW_SPEC = pl.BlockSpec((_ROW_BLK, HID), lambda i: (i, 0))
_DEG_SPEC = pl.BlockSpec((2, _ROW_BLK, _DW), lambda i: (0, i, 0))
_W_SPEC = pl.BlockSpec((HID, HID), lambda i: (0, 0))
_B_SPEC = pl.BlockSpec((1, HID), lambda i: (0, 0))
_M_SPEC = pl.BlockSpec((2, _ROW_BLK, HALF), lambda i: (0, i, 0))
_M_SHAPE = jax.ShapeDtypeStruct((2, N, HALF), jnp.float32)


def _enc_call(sc, x, W_enc, b_enc, W0, b0):
    return pl.pallas_call(
        _enc_body,
        grid=(_GRID,),
        in_specs=[_SC_SPEC, _ROW_SPEC, _W_SPEC, _B_SPEC, _W_SPEC, _B_SPEC],
        out_specs=_M_SPEC,
        out_shape=_M_SHAPE,
    )(sc, x, W_enc, b_enc, W0, b0)


def _mid_call(sc, agg, deg, W, b):
    return pl.pallas_call(
        _mid_body,
        grid=(_GRID,),
        in_specs=[_SC_SPEC, _ROW_SPEC, _DEG_SPEC, _W_SPEC, _B_SPEC],
        out_specs=_M_SPEC,
        out_shape=_M_SHAPE,
    )(sc, agg, deg, W, b)


def _head_call(sc, agg, deg, W_head, b_head):
    return pl.pallas_call(
        _head_body,
        grid=(_GRID,),
        in_specs=[_SC_SPEC, _ROW_SPEC, _DEG_SPEC,
                  pl.BlockSpec((HID, D_OUT), lambda i: (0, 0)),
                  pl.BlockSpec((1, D_OUT), lambda i: (0, 0))],
        out_specs=pl.BlockSpec((_ROW_BLK, D_OUT), lambda i: (i, 0)),
        out_shape=jax.ShapeDtypeStruct((N, D_OUT), jnp.float32),
    )(sc, agg, deg, W_head, b_head)


def _zero_fill(zbuf, nrows, ncols):
    zero16 = jnp.zeros((_LANES,), jnp.float32)

    def zrow(i, carry):
        for j in range(ncols // _LANES):
            zbuf[i, pl.ds(j * _LANES, _LANES)] = zero16
        return carry

    lax.fori_loop(0, nrows, zrow, 0)


def _zero_rows_of(zbuf, dstref):
    def zero_rows(r0, nrows):
        for q in range(nrows // _ZROWS):
            pltpu.sync_copy(zbuf, dstref.at[pl.ds(r0 + q * _ZROWS, _ZROWS)])
        rem = nrows % _ZROWS
        if rem:
            pltpu.sync_copy(zbuf.at[pl.ds(0, rem)],
                            dstref.at[pl.ds(r0 + nrows - rem, rem)])

    return zero_rows


def _for_tile_chunks(s, fn):
    for t in range(_OUT_PER_TILE):
        fn((s * _OUT_PER_TILE + t) * _OUT_BLK, _OUT_BLK)

    @pl.when(s == 0)
    def _():
        fn(_NS * _OUT_PER_TILE * _OUT_BLK, _TAIL_ROWS)


def _agg_body(m_hbm, src_hbm, dst_hbm, out_hbm,
              sidxb, didxb, dstv, gbufa, gbufb, zbuf, aggs,
              sema, semb):
    c = lax.axis_index("c")
    s = lax.axis_index("s")

    _zero_fill(zbuf, _ZROWS, HALF)
    _for_tile_chunks(s, _zero_rows_of(zbuf, aggs))
    plsc.subcore_barrier()

    ebase = s * _E_PER_TILE
    coff = c * N
    blen = _NB * _CHUNK

    def batch(b, carry):
        boff = ebase + b * blen
        pltpu.sync_copy(src_hbm.at[pl.ds(boff, blen)], sidxb)
        pltpu.sync_copy(dst_hbm.at[pl.ds(boff, blen)], didxb)
        for i in range(blen // _LANES):
            sl = pl.ds(i * _LANES, _LANES)
            sidxb[sl] = sidxb[sl] + coff

        def sg(j, gbuf, sem):
            idx = sidxb.at[pl.ds(j * _CHUNK, _CHUNK)]
            return pltpu.async_copy(m_hbm.at[idx], gbuf, sem)

        def scat(j, gbuf, sem):
            idx = sidxb.at[pl.ds(j * _CHUNK, _CHUNK)]
            pltpu.make_async_copy(m_hbm.at[idx], gbuf, sem).wait()
            for q in range(_CHUNK // _LANES):
                dstv[pl.ds(q * _LANES, _LANES)] = (
                    didxb[pl.ds(j * _CHUNK + q * _LANES, _LANES)])
            pltpu.sync_copy(gbuf, aggs.at[dstv], add=True)

        sg(0, gbufa, sema)
        for p in range(_NB // 2):
            sg(2 * p + 1, gbufb, semb)
            scat(2 * p, gbufa, sema)
            sg(2 * p + 2, gbufa, sema)
            scat(2 * p + 1, gbufb, semb)
        scat(_NB - 1, gbufa, sema)
        return carry

    lax.fori_loop(0, _N_CHUNKS // _NB, batch, 0)
    plsc.subcore_barrier()

    def write_rows(r0, nrows):
        pltpu.sync_copy(aggs.at[pl.ds(r0, nrows)],
                        out_hbm.at[pl.ds(r0, nrows), pl.ds(c * HALF, HALF)])

    _for_tile_chunks(s, write_rows)


_DCHUNK = 200
_E_PER_WORKER = E // (_NC * _NS)
_D_CHUNKS = _E_PER_WORKER // _DCHUNK


def _deg_body(dst_hbm, deg_hbm, dstv, dstv2, onesb, zbuf, degs, sem, semb):
    c = lax.axis_index("c")
    s = lax.axis_index("s")

    _zero_fill(zbuf, _ZROWS, _DW)
    one16 = jnp.ones((_LANES,), jnp.float32)

    def ofill(i, carry):
        for j in range(_DW // _LANES):
            onesb[i, pl.ds(j * _LANES, _LANES)] = one16
        return carry

    lax.fori_loop(0, _DCHUNK, ofill, 0)

    _for_tile_chunks(s, _zero_rows_of(zbuf, degs))
    plsc.subcore_barrier()

    ebase = (c * _NS + s) * _E_PER_WORKER

    def load_idx(k, buf, sm):
        return pltpu.async_copy(
            dst_hbm.at[pl.ds(ebase + k * _DCHUNK, _DCHUNK)], buf, sm)

    def wait_idx(buf, sm):
        pltpu.make_async_copy(dst_hbm.at[pl.ds(ebase, _DCHUNK)], buf, sm).wait()

    load_idx(0, dstv, sem)

    def steppair(k2, carry):
        load_idx(2 * k2 + 1, dstv2, semb)
        wait_idx(dstv, sem)
        pltpu.sync_copy(onesb, degs.at[dstv], add=True)
        load_idx(2 * k2 + 2, dstv, sem)
        wait_idx(dstv2, semb)
        pltpu.sync_copy(onesb, degs.at[dstv2], add=True)
        return carry

    lax.fori_loop(0, _D_CHUNKS // 2, steppair, 0)
    wait_idx(dstv, sem)
    pltpu.sync_copy(onesb, degs.at[dstv], add=True)
    plsc.subcore_barrier()

    def write_rows(r0, nrows):
        pltpu.sync_copy(degs.at[pl.ds(r0, nrows)],
                        deg_hbm.at[pl.ds(c * N + r0, nrows)])

    _for_tile_chunks(s, write_rows)


@functools.lru_cache(maxsize=None)
def _make_agg():
    mesh = plsc.VectorSubcoreMesh(core_axis_name="c", subcore_axis_name="s",
                                  num_cores=_NC, num_subcores=_NS)
    scratch = [
        pltpu.VMEM((_NB * _CHUNK,), jnp.int32),
        pltpu.VMEM((_NB * _CHUNK,), jnp.int32),
        pltpu.VMEM((_CHUNK,), jnp.int32),
        pltpu.VMEM((_CHUNK, HALF), jnp.float32),
        pltpu.VMEM((_CHUNK, HALF), jnp.float32),
        pltpu.VMEM((_ZROWS, HALF), jnp.float32),
        pltpu.VMEM_SHARED((N, HALF), jnp.float32),
        pltpu.SemaphoreType.DMA,
        pltpu.SemaphoreType.DMA,
    ]
    return pl.kernel(_agg_body,
                     out_type=jax.ShapeDtypeStruct((N, HID), jnp.float32),
                     mesh=mesh, scratch_types=tuple(scratch))


@functools.lru_cache(maxsize=None)
def _make_deg():
    mesh = plsc.VectorSubcoreMesh(core_axis_name="c", subcore_axis_name="s",
                                  num_cores=_NC, num_subcores=_NS)
    scratch = [
        pltpu.VMEM((_DCHUNK,), jnp.int32),
        pltpu.VMEM((_DCHUNK,), jnp.int32),
        pltpu.VMEM((_DCHUNK, _DW), jnp.float32),
        pltpu.VMEM((_ZROWS, _DW), jnp.float32),
        pltpu.VMEM_SHARED((N, _DW), jnp.float32),
        pltpu.SemaphoreType.DMA,
        pltpu.SemaphoreType.DMA,
    ]
    return pl.kernel(_deg_body,
                     out_type=jax.ShapeDtypeStruct((2 * N, _DW), jnp.float32),
                     mesh=mesh, scratch_types=tuple(scratch))


def kernel(x, edge_index, c_param, W_enc, b_enc, W0, b0, W1, b1, W2, b2,
           W_head, b_head):
    c = jnp.abs(c_param) + 1e-5
    sc = jnp.sqrt(c).reshape(1, 1).astype(jnp.float32)

    ei = edge_index.astype(jnp.int32)
    src = ei[0]
    dst = ei[1]

    deg = _make_deg()(dst).reshape(2, N, _DW)
    m = _enc_call(sc, x, W_enc, b_enc.reshape(1, HID), W0, b0.reshape(1, HID))
    agg = _make_agg()(m.reshape(2 * N, HALF), src, dst)

    for (W, b) in ((W1, b1), (W2, b2)):
        m = _mid_call(sc, agg, deg, W, b.reshape(1, HID))
        agg = _make_agg()(m.reshape(2 * N, HALF), src, dst)

    return _head_call(sc, agg, deg, W_head, b_head.reshape(1, D_OUT))


# speedup vs baseline: 7.3873x; 1.0044x over previous
import functools

import jax
import jax.numpy as jnp
from jax import lax
from jax.experimental import pallas as pl
from jax.experimental.pallas import tpu as pltpu
from jax.experimental.pallas import tpu_sc as plsc

EPS = 1e-7

N = 10000
E = 160000
HID = 256
HALF = 128
D_OUT = 128

_NC = 2
_NS = 16
_LANES = 16

_ROW_BLK = 2048
_CHUNK = 80
_NB = 25
_E_PER_TILE = E // _NS
_N_CHUNKS = _E_PER_TILE // _CHUNK
_OUT_BLK = 208
_OUT_PER_TILE = 3
_TAIL_ROWS = N - _NS * _OUT_PER_TILE * _OUT_BLK
_ZROWS = 16
_DW = 128


def _row_norm(v):
    return jnp.sqrt(jnp.sum(v * v, axis=-1, keepdims=True))


def _exp_map(v, sc):
    n = jnp.maximum(_row_norm(v), EPS)
    return jnp.tanh(sc * n) * v / (sc * n)


def _arctanh(x):
    return 0.5 * jnp.log((1.0 + x) / (1.0 - x))


def _log_map(y, sc):
    n = jnp.maximum(_row_norm(y), EPS)
    scn = jnp.clip(sc * n, EPS, 1.0 - 1e-5)
    return _arctanh(scn) * y / (sc * n)


def _split_store(m_ref, m):
    m_ref[0] = m[:, :HALF]
    m_ref[1] = m[:, HALF:]


def _enc_body(sc_ref, x_ref, we_ref, be_ref, w0_ref, b0_ref, m_ref):
    sc = sc_ref[0, 0]
    y = jnp.dot(x_ref[...], we_ref[...],
                preferred_element_type=jnp.float32) + be_ref[...]
    ht = _log_map(_exp_map(y, sc), sc)
    _split_store(m_ref, jnp.dot(ht, w0_ref[...],
                                preferred_element_type=jnp.float32) + b0_ref[...])


def _deg_col(deg_ref):
    d = deg_ref[0] + deg_ref[1]
    return jnp.maximum(d[:, 0:1], 1.0)


def _mid_body(sc_ref, agg_ref, deg_ref, w_ref, b_ref, m_ref):
    sc = sc_ref[0, 0]
    deg = _deg_col(deg_ref)
    z = agg_ref[...].astype(jnp.float32) / deg
    h = _exp_map(_log_map(_exp_map(z, sc), sc), sc)
    ht = _log_map(h, sc)
    _split_store(m_ref, jnp.dot(ht, w_ref[...],
                                preferred_element_type=jnp.float32) + b_ref[...])


def _head_body(sc_ref, agg_ref, deg_ref, w_ref, b_ref, o_ref):
    sc = sc_ref[0, 0]
    deg = _deg_col(deg_ref)
    z = agg_ref[...].astype(jnp.float32) / deg
    h = _exp_map(_log_map(_exp_map(z, sc), sc), sc)
    ht = _log_map(h, sc)
    o_ref[...] = jnp.dot(ht, w_ref[...],
                         preferred_element_type=jnp.float32) + b_ref[...]


_GRID = -(-N // _ROW_BLK)

_SC_SPEC = pl.BlockSpec((1, 1), lambda i: (0, 0))
_ROW_SPEC = pl.BlockSpec((_ROW_BLK, HID), lambda i: (i, 0))
_DEG_SPEC = pl.BlockSpec((2, _ROW_BLK, _DW), lambda i: (0, i, 0))
_W_SPEC = pl.BlockSpec((HID, HID), lambda i: (0, 0))
_B_SPEC = pl.BlockSpec((1, HID), lambda i: (0, 0))
_M_SPEC = pl.BlockSpec((2, _ROW_BLK, HALF), lambda i: (0, i, 0))
_M_SHAPE = jax.ShapeDtypeStruct((2, N, HALF), jnp.float32)


def _enc_call(sc, x, W_enc, b_enc, W0, b0):
    return pl.pallas_call(
        _enc_body,
        grid=(_GRID,),
        in_specs=[_SC_SPEC, _ROW_SPEC, _W_SPEC, _B_SPEC, _W_SPEC, _B_SPEC],
        out_specs=_M_SPEC,
        out_shape=_M_SHAPE,
    )(sc, x, W_enc, b_enc, W0, b0)


def _mid_call(sc, agg, deg, W, b):
    return pl.pallas_call(
        _mid_body,
        grid=(_GRID,),
        in_specs=[_SC_SPEC, _ROW_SPEC, _DEG_SPEC, _W_SPEC, _B_SPEC],
        out_specs=_M_SPEC,
        out_shape=_M_SHAPE,
    )(sc, agg, deg, W, b)


def _head_call(sc, agg, deg, W_head, b_head):
    return pl.pallas_call(
        _head_body,
        grid=(_GRID,),
        in_specs=[_SC_SPEC, _ROW_SPEC, _DEG_SPEC,
                  pl.BlockSpec((HID, D_OUT), lambda i: (0, 0)),
                  pl.BlockSpec((1, D_OUT), lambda i: (0, 0))],
        out_specs=pl.BlockSpec((_ROW_BLK, D_OUT), lambda i: (i, 0)),
        out_shape=jax.ShapeDtypeStruct((N, D_OUT), jnp.float32),
    )(sc, agg, deg, W_head, b_head)


def _zero_fill(zbuf, nrows, ncols):
    zero16 = jnp.zeros((_LANES,), jnp.float32)

    def zrow(i, carry):
        for j in range(ncols // _LANES):
            zbuf[i, pl.ds(j * _LANES, _LANES)] = zero16
        return carry

    lax.fori_loop(0, nrows, zrow, 0)


def _zero_rows_of(zbuf, dstref):
    def zero_rows(r0, nrows):
        for q in range(nrows // _ZROWS):
            pltpu.sync_copy(zbuf, dstref.at[pl.ds(r0 + q * _ZROWS, _ZROWS)])
        rem = nrows % _ZROWS
        if rem:
            pltpu.sync_copy(zbuf.at[pl.ds(0, rem)],
                            dstref.at[pl.ds(r0 + nrows - rem, rem)])

    return zero_rows


def _for_tile_chunks(s, fn):
    for t in range(_OUT_PER_TILE):
        fn((s * _OUT_PER_TILE + t) * _OUT_BLK, _OUT_BLK)

    @pl.when(s == 0)
    def _():
        fn(_NS * _OUT_PER_TILE * _OUT_BLK, _TAIL_ROWS)


def _agg_body(m_hbm, src_hbm, dst_hbm, out_hbm,
              sidxb, didxb, dstv, gbufa, gbufb, zbuf, aggs,
              sema, semb):
    c = lax.axis_index("c")
    s = lax.axis_index("s")

    _zero_fill(zbuf, _ZROWS, HALF)
    _for_tile_chunks(s, _zero_rows_of(zbuf, aggs))
    plsc.subcore_barrier()

    ebase = s * _E_PER_TILE
    coff = c * N
    blen = _NB * _CHUNK

    def batch(b, carry):
        boff = ebase + b * blen
        pltpu.sync_copy(src_hbm.at[pl.ds(boff, blen)], sidxb)
        pltpu.sync_copy(dst_hbm.at[pl.ds(boff, blen)], didxb)
        for i in range(blen // _LANES):
            sl = pl.ds(i * _LANES, _LANES)
            sidxb[sl] = sidxb[sl] + coff

        def sg(j, gbuf, sem):
            idx = sidxb.at[pl.ds(j * _CHUNK, _CHUNK)]
            return pltpu.async_copy(m_hbm.at[idx], gbuf, sem)

        def scat(j, gbuf, sem):
            idx = sidxb.at[pl.ds(j * _CHUNK, _CHUNK)]
            pltpu.make_async_copy(m_hbm.at[idx], gbuf, sem).wait()
            didx = didxb.at[pl.ds(j * _CHUNK, _CHUNK)]
            pltpu.sync_copy(gbuf, aggs.at[didx], add=True)

        sg(0, gbufa, sema)
        for p in range(_NB // 2):
            sg(2 * p + 1, gbufb, semb)
            scat(2 * p, gbufa, sema)
            sg(2 * p + 2, gbufa, sema)
            scat(2 * p + 1, gbufb, semb)
        scat(_NB - 1, gbufa, sema)
        return carry

    lax.fori_loop(0, _N_CHUNKS // _NB, batch, 0)
    plsc.subcore_barrier()

    def write_rows(r0, nrows):
        pltpu.sync_copy(aggs.at[pl.ds(r0, nrows)],
                        out_hbm.at[pl.ds(r0, nrows), pl.ds(c * HALF, HALF)])

    _for_tile_chunks(s, write_rows)


_DCHUNK = 200
_E_PER_WORKER = E // (_NC * _NS)
_D_CHUNKS = _E_PER_WORKER // _DCHUNK


def _deg_body(dst_hbm, deg_hbm, dstv, dstv2, onesb, zbuf, degs, sem, semb):
    c = lax.axis_index("c")
    s = lax.axis_index("s")

    _zero_fill(zbuf, _ZROWS, _DW)
    one16 = jnp.ones((_LANES,), jnp.float32)

    def ofill(i, carry):
        for j in range(_DW // _LANES):
            onesb[i, pl.ds(j * _LANES, _LANES)] = one16
        return carry

    lax.fori_loop(0, _DCHUNK, ofill, 0)

    _for_tile_chunks(s, _zero_rows_of(zbuf, degs))
    plsc.subcore_barrier()

    ebase = (c * _NS + s) * _E_PER_WORKER

    def load_idx(k, buf, sm):
        return pltpu.async_copy(
            dst_hbm.at[pl.ds(ebase + k * _DCHUNK, _DCHUNK)], buf, sm)

    def wait_idx(buf, sm):
        pltpu.make_async_copy(dst_hbm.at[pl.ds(ebase, _DCHUNK)], buf, sm).wait()

    load_idx(0, dstv, sem)

    def steppair(k2, carry):
        load_idx(2 * k2 + 1, dstv2, semb)
        wait_idx(dstv, sem)
        pltpu.sync_copy(onesb, degs.at[dstv], add=True)
        load_idx(2 * k2 + 2, dstv, sem)
        wait_idx(dstv2, semb)
        pltpu.sync_copy(onesb, degs.at[dstv2], add=True)
        return carry

    lax.fori_loop(0, _D_CHUNKS // 2, steppair, 0)
    wait_idx(dstv, sem)
    pltpu.sync_copy(onesb, degs.at[dstv], add=True)
    plsc.subcore_barrier()

    def write_rows(r0, nrows):
        pltpu.sync_copy(degs.at[pl.ds(r0, nrows)],
                        deg_hbm.at[pl.ds(c * N + r0, nrows)])

    _for_tile_chunks(s, write_rows)


@functools.lru_cache(maxsize=None)
def _make_agg():
    mesh = plsc.VectorSubcoreMesh(core_axis_name="c", subcore_axis_name="s",
                                  num_cores=_NC, num_subcores=_NS)
    scratch = [
        pltpu.VMEM((_NB * _CHUNK,), jnp.int32),
        pltpu.VMEM((_NB * _CHUNK,), jnp.int32),
        pltpu.VMEM((_CHUNK,), jnp.int32),
        pltpu.VMEM((_CHUNK, HALF), jnp.float32),
        pltpu.VMEM((_CHUNK, HALF), jnp.float32),
        pltpu.VMEM((_ZROWS, HALF), jnp.float32),
        pltpu.VMEM_SHARED((N, HALF), jnp.float32),
        pltpu.SemaphoreType.DMA,
        pltpu.SemaphoreType.DMA,
    ]
    return pl.kernel(_agg_body,
                     out_type=jax.ShapeDtypeStruct((N, HID), jnp.float32),
                     mesh=mesh, scratch_types=tuple(scratch))


@functools.lru_cache(maxsize=None)
def _make_deg():
    mesh = plsc.VectorSubcoreMesh(core_axis_name="c", subcore_axis_name="s",
                                  num_cores=_NC, num_subcores=_NS)
    scratch = [
        pltpu.VMEM((_DCHUNK,), jnp.int32),
        pltpu.VMEM((_DCHUNK,), jnp.int32),
        pltpu.VMEM((_DCHUNK, _DW), jnp.float32),
        pltpu.VMEM((_ZROWS, _DW), jnp.float32),
        pltpu.VMEM_SHARED((N, _DW), jnp.float32),
        pltpu.SemaphoreType.DMA,
        pltpu.SemaphoreType.DMA,
    ]
    return pl.kernel(_deg_body,
                     out_type=jax.ShapeDtypeStruct((2 * N, _DW), jnp.float32),
                     mesh=mesh, scratch_types=tuple(scratch))


def kernel(x, edge_index, c_param, W_enc, b_enc, W0, b0, W1, b1, W2, b2,
           W_head, b_head):
    c = jnp.abs(c_param) + 1e-5
    sc = jnp.sqrt(c).reshape(1, 1).astype(jnp.float32)

    ei = edge_index.astype(jnp.int32)
    src = ei[0]
    dst = ei[1]

    deg = _make_deg()(dst).reshape(2, N, _DW)
    m = _enc_call(sc, x, W_enc, b_enc.reshape(1, HID), W0, b0.reshape(1, HID))
    agg = _make_agg()(m.reshape(2 * N, HALF), src, dst)

    for (W, b) in ((W1, b1), (W2, b2)):
        m = _mid_call(sc, agg, deg, W, b.reshape(1, HID))
        agg = _make_agg()(m.reshape(2 * N, HALF), src, dst)

    return _head_call(sc, agg, deg, W_head, b_head.reshape(1, D_OUT))
